# speedup vs baseline: 2.3068x; 2.3068x over previous
import functools

import jax
import jax.numpy as jnp
from jax import lax
from jax.experimental import pallas as pl
from jax.experimental.pallas import tpu as pltpu
from jax.experimental.pallas import tpu_sc as plsc

MAX_LEN = 8192
D = 1024
B = 4 * 8192
NC, NS = 2, 16
NW = NC * NS
BPW = B // NW
CH = 32
NCHUNK = BPW // CH
NB = 2

_mesh = plsc.VectorSubcoreMesh(core_axis_name="c", subcore_axis_name="s")


@functools.partial(
    pl.kernel,
    mesh=_mesh,
    out_type=jax.ShapeDtypeStruct((B, D), jnp.float32),
    scratch_types=[
        pltpu.VMEM((BPW,), jnp.int32),
        pltpu.VMEM((NB, CH, D), jnp.float32),
        pltpu.SemaphoreType.DMA,
        pltpu.SemaphoreType.DMA,
    ],
)
def _gather_kernel(pos_hbm, table_hbm, out_hbm, idx_v, buf_v, sem0, sem1):
    wid = lax.axis_index("s") * NC + lax.axis_index("c")
    base = wid * BPW
    pltpu.sync_copy(pos_hbm.at[pl.ds(base, BPW)], idx_v)
    sems = (sem0, sem1)

    def gather_start(g, b):
        pltpu.async_copy(
            table_hbm.at[idx_v.at[pl.ds(g * CH, CH)]], buf_v.at[b], sems[b])

    def gather_wait(g, b):
        pltpu.make_async_copy(
            table_hbm.at[idx_v.at[pl.ds(g * CH, CH)]], buf_v.at[b],
            sems[b]).wait()

    def copy_out(g, b):
        pltpu.sync_copy(buf_v.at[b], out_hbm.at[pl.ds(base + g * CH, CH)])

    gather_start(0, 0)

    def body(it, carry):
        i = it * NB
        for b in range(NB):
            g = i + b
            gather_wait(g, b)
            gather_start(g + 1, (b + 1) % NB)
            copy_out(g, b)
        return carry

    lax.fori_loop(0, (NCHUNK - NB) // NB, body, 0)

    for b in range(NB):
        g = NCHUNK - NB + b
        gather_wait(g, b)
        if b + 1 < NB:
            gather_start(g + 1, b + 1)
        copy_out(g, b)


def kernel(positions, table):
    pos = jnp.clip(positions, 0, MAX_LEN - 1).reshape(B)
    out = _gather_kernel(pos, table)
    return out.reshape(positions.shape[0], positions.shape[1], D)


# speedup vs baseline: 2.3775x; 1.0307x over previous
import functools

import jax
import jax.numpy as jnp
from jax import lax
from jax.experimental import pallas as pl
from jax.experimental.pallas import tpu as pltpu
from jax.experimental.pallas import tpu_sc as plsc

MAX_LEN = 8192
D = 1024
B = 4 * 8192
NC, NS = 2, 16
NW = NC * NS
BPW = B // NW
CH = 16
NCHUNK = BPW // CH
NB = 4
LOOKAHEAD = 2

_mesh = plsc.VectorSubcoreMesh(core_axis_name="c", subcore_axis_name="s")


@functools.partial(
    pl.kernel,
    mesh=_mesh,
    out_type=jax.ShapeDtypeStruct((B, D), jnp.float32),
    scratch_types=[
        pltpu.VMEM((BPW,), jnp.int32),
        pltpu.VMEM((NB, CH, D), jnp.float32),
    ]
    + [pltpu.SemaphoreType.DMA] * (2 * NB),
)
def _gather_kernel(pos_hbm, table_hbm, out_hbm, idx_v, buf_v, *sems):
    gsems, osems = sems[:NB], sems[NB:]
    wid = lax.axis_index("s") * NC + lax.axis_index("c")
    base = wid * BPW
    pltpu.sync_copy(pos_hbm.at[pl.ds(base, BPW)], idx_v)

    def gather_start(g, b):
        pltpu.async_copy(
            table_hbm.at[idx_v.at[pl.ds(g * CH, CH)]], buf_v.at[b], gsems[b])

    def gather_wait(g, b):
        pltpu.make_async_copy(
            table_hbm.at[idx_v.at[pl.ds(g * CH, CH)]], buf_v.at[b],
            gsems[b]).wait()

    def out_start(g, b):
        pltpu.async_copy(
            buf_v.at[b], out_hbm.at[pl.ds(base + g * CH, CH)], osems[b])

    def out_wait(g, b):
        pltpu.make_async_copy(
            buf_v.at[b], out_hbm.at[pl.ds(base + g * CH, CH)], osems[b]).wait()

    def step(g, b, head, tail):
        gather_wait(g, b)
        out_start(g, b)
        if head:
            bn = (b + LOOKAHEAD) % NB
            if tail:
                out_wait(g + LOOKAHEAD - NB, bn)
            gather_start(g + LOOKAHEAD, bn)

    for g in range(LOOKAHEAD):
        gather_start(g, g % NB)
    for g in range(NB - LOOKAHEAD):
        step(g, g % NB, head=True, tail=False)

    def body(it, carry):
        i = (NB - LOOKAHEAD) + it * NB
        for j in range(NB):
            step(i + j, (NB - LOOKAHEAD + j) % NB, head=True, tail=True)
        return carry

    lax.fori_loop(0, (NCHUNK - NB) // NB, body, 0)

    for g in range(NCHUNK - LOOKAHEAD, NCHUNK):
        step(g, g % NB, head=False, tail=False)
    for g in range(NCHUNK - NB, NCHUNK):
        out_wait(g, g % NB)


def kernel(positions, table):
    pos = jnp.clip(positions, 0, MAX_LEN - 1).reshape(B)
    out = _gather_kernel(pos, table)
    return out.reshape(positions.shape[0], positions.shape[1], D)
